# baseline (speedup 1.0000x reference)
import functools

import jax
import jax.numpy as jnp
import numpy as np
from jax.experimental import pallas as pl

_NUM_TIME_MASKS = 10
_NUM_FREQ_MASKS = 2
_TIME_MASK_RATIO = 0.05
_MAX_FREQ_MASK_SIZE = 27

_TB = 8


def _mask_constants(frame: int, n_mels: int):
    rng = np.random.default_rng(0)
    f = int(rng.integers(0, _MAX_FREQ_MASK_SIZE + 1))
    f0 = rng.integers(0, n_mels - f, size=(_NUM_FREQ_MASKS,))
    fsegs = [(int(s), int(s) + f) for s in sorted(f0)] if f > 0 else []
    max_t = int(np.floor(_TIME_MASK_RATIO * frame))
    t = int(rng.integers(0, max_t + 1))
    t0 = rng.integers(0, frame - t, size=(_NUM_TIME_MASKS,))
    segs = [(int(s), int(s) + t) for s in sorted(t0)] if t > 0 else []
    return segs, fsegs


def _full_block_runs(segs, tb):
    runs = []
    for s, e in segs:
        k0 = -(-s // tb)
        k1 = (e - tb) // tb
        if k0 <= k1 and k0 >= 1:
            runs.append((k0, k1))
    return runs


def _mask_kernel(x_ref, o_ref, *, segs, fsegs, tb, n_mels):
    j = pl.program_id(0)
    rows = j * tb + jax.lax.broadcasted_iota(jnp.int32, (tb, 1), 0)
    t_masked = functools.reduce(
        jnp.logical_or,
        [(rows >= s) & (rows < e) for s, e in segs],
        jnp.zeros((tb, 1), jnp.bool_),
    )
    cols = jax.lax.broadcasted_iota(jnp.int32, (1, n_mels), 1)
    f_masked = functools.reduce(
        jnp.logical_or,
        [(cols >= s) & (cols < e) for s, e in fsegs],
        jnp.zeros((1, n_mels), jnp.bool_),
    )
    m = jnp.where(t_masked | f_masked, 0.0, 1.0)[None, :, :]
    o_ref[...] = x_ref[...] * m


def kernel(x):
    b, frame, n_mels = x.shape
    segs, fsegs = _mask_constants(frame, n_mels)
    runs = _full_block_runs(segs, _TB)

    def in_map(j):
        t = j
        for k0, k1 in runs:
            t = jnp.where((j >= k0) & (j <= k1), k0 - 1, t)
        return (0, t, 0)

    body = functools.partial(
        _mask_kernel, segs=segs, fsegs=fsegs, tb=_TB, n_mels=n_mels
    )
    return pl.pallas_call(
        body,
        grid=(frame // _TB,),
        in_specs=[pl.BlockSpec((b, _TB, n_mels), in_map)],
        out_specs=pl.BlockSpec((b, _TB, n_mels), lambda j: (0, j, 0)),
        out_shape=jax.ShapeDtypeStruct(x.shape, x.dtype),
    )(x)


# speedup vs baseline: 2.3647x; 2.3647x over previous
import jax
import jax.numpy as jnp
import numpy as np
from jax.experimental import pallas as pl
from jax.experimental.pallas import tpu as pltpu

_NUM_TIME_MASKS = 10
_NUM_FREQ_MASKS = 2
_TIME_MASK_RATIO = 0.05
_MAX_FREQ_MASK_SIZE = 27

_BB = 8


def _mask_constants(frame: int, n_mels: int):
    rng = np.random.default_rng(0)
    f = int(rng.integers(0, _MAX_FREQ_MASK_SIZE + 1))
    f0 = rng.integers(0, n_mels - f, size=(_NUM_FREQ_MASKS,))
    fmask = np.ones((n_mels,), np.float32)
    if f > 0:
        for s in f0:
            fmask[s : s + f] = 0.0
    max_t = int(np.floor(_TIME_MASK_RATIO * frame))
    t = int(rng.integers(0, max_t + 1))
    t0 = rng.integers(0, frame - t, size=(_NUM_TIME_MASKS,))
    tmask = np.ones((frame,), np.float32)
    segs = []
    if t > 0:
        for s in sorted(int(v) for v in t0):
            tmask[s : s + t] = 0.0
            segs.append((s, s + t))
    runs, prev = [], 0
    for s, e in segs:
        if s > prev:
            runs.append((prev, s))
        prev = max(prev, e)
    if prev < frame:
        runs.append((prev, frame))
    plane = tmask[:, None] * fmask[None, :]
    return runs, plane


def kernel(x):
    b, frame, n_mels = x.shape
    runs, plane = _mask_constants(frame, n_mels)
    mask = jnp.asarray(plane)[None, :, :]
    nsteps = b // _BB

    def body(x_hbm, m_ref, o_ref, buf, sems):
        i = pl.program_id(0)
        slot = jax.lax.rem(i, 2)

        def copies(step, slot):
            for r0, r1 in runs:
                yield pltpu.make_async_copy(
                    x_hbm.at[pl.ds(step * _BB, _BB), pl.ds(r0, r1 - r0), :],
                    buf.at[slot, :, pl.ds(r0, r1 - r0), :],
                    sems.at[slot],
                )

        @pl.when(i == 0)
        def _():
            for c in copies(0, 0):
                c.start()

        @pl.when(i + 1 < nsteps)
        def _():
            for c in copies(i + 1, 1 - slot):
                c.start()

        for c in copies(i, slot):
            c.wait()

        o_ref[...] = jnp.where(m_ref[...] != 0.0, buf[slot], 0.0)

    return pl.pallas_call(
        body,
        grid=(nsteps,),
        in_specs=[
            pl.BlockSpec(memory_space=pl.ANY),
            pl.BlockSpec((1, frame, n_mels), lambda i: (0, 0, 0)),
        ],
        out_specs=pl.BlockSpec((_BB, frame, n_mels), lambda i: (i, 0, 0)),
        out_shape=jax.ShapeDtypeStruct(x.shape, x.dtype),
        scratch_shapes=[
            pltpu.VMEM((2, _BB, frame, n_mels), jnp.float32),
            pltpu.SemaphoreType.DMA((2,)),
        ],
    )(x, mask)
